# baseline (speedup 1.0000x reference)
import functools
import jax
import jax.numpy as jnp
from jax import lax
from jax.experimental import pallas as pl
from jax.experimental.pallas import tpu as pltpu
from jax.experimental.pallas import tpu_sc as plsc

B, N, F = 4, 8192, 2048
BSC = 2
NW = 32
WPB = NW // BSC
FW = F // WPB
R = 128
NT = N // R
NV = FW // 16

_mesh = plsc.VectorSubcoreMesh(core_axis_name="c", subcore_axis_name="s")


@functools.partial(
    pl.kernel,
    mesh=_mesh,
    out_type=jax.ShapeDtypeStruct((BSC, N, F), jnp.float32),
    scratch_types=[
        pltpu.VMEM((2, R, FW), jnp.float32),
        pltpu.SemaphoreType.DMA,
        pltpu.SemaphoreType.DMA,
        pltpu.SemaphoreType.DMA,
        pltpu.SemaphoreType.DMA,
    ],
)
def _sc_cumsum(x_hbm, out_hbm, buf, lsem0, lsem1, ssem0, ssem1):
    wid = lax.axis_index("s") * 2 + lax.axis_index("c")
    b = wid // WPB
    f0 = (wid % WPB) * FW
    lsems = [lsem0, lsem1]
    ssems = [ssem0, ssem1]

    def load_copy(t, s):
        return pltpu.make_async_copy(
            x_hbm.at[b, pl.ds(t * R, R), pl.ds(f0, FW)],
            buf.at[s],
            lsems[s],
        )

    def store_copy(t, s):
        return pltpu.make_async_copy(
            buf.at[s],
            out_hbm.at[b, pl.ds(t * R, R), pl.ds(f0, FW)],
            ssems[s],
        )

    load_copy(0, 0).start()
    load_copy(1, 1).start()

    def phase(t, s, carry):
        load_copy(t, s).wait()

        def row(r, acc):
            new = []
            for j in range(NV):
                v = acc[j] + buf[s, r, pl.ds(16 * j, 16)]
                buf[s, r, pl.ds(16 * j, 16)] = v
                new.append(v)
            return tuple(new)

        carry = lax.fori_loop(0, R, row, carry, unroll=2)
        store_copy(t, s).start()
        store_copy(t, s).wait()

        @pl.when(t + 2 < NT)
        def _():
            load_copy(t + 2, s).start()

        return carry

    def two(i, carry):
        t = i * 2
        carry = phase(t, 0, carry)
        carry = phase(t + 1, 1, carry)
        return carry

    zeros = tuple(jnp.zeros((16,), jnp.float32) for _ in range(NV))
    lax.fori_loop(0, NT // 2, two, zeros)


S = 256


def _tc_body(x_ref, o_ref, carry_ref):
    s = pl.program_id(1)

    @pl.when(s == 0)
    def _():
        carry_ref[...] = jnp.zeros_like(carry_ref)

    x = x_ref[0]
    r = jax.lax.broadcasted_iota(jnp.int32, (S, S), 0)
    c = jax.lax.broadcasted_iota(jnp.int32, (S, S), 1)
    tri = (c <= r).astype(jnp.float32)
    acc = jnp.dot(tri, x, preferred_element_type=jnp.float32)
    acc = acc + carry_ref[...]
    o_ref[0] = acc
    carry_ref[...] = acc[S - 1 : S, :]


def _tc_cumsum(x):
    return pl.pallas_call(
        _tc_body,
        grid=(B - BSC, N // S),
        in_specs=[pl.BlockSpec((1, S, F), lambda b, s: (b + BSC, s, 0))],
        out_specs=pl.BlockSpec((1, S, F), lambda b, s: (b, s, 0)),
        out_shape=jax.ShapeDtypeStruct((B - BSC, N, F), jnp.float32),
        scratch_shapes=[pltpu.VMEM((1, F), jnp.float32)],
    )(x)


def kernel(x):
    out_sc = _sc_cumsum(x)
    out_tc = _tc_cumsum(x)
    return jnp.concatenate([out_sc, out_tc], axis=0)


# speedup vs baseline: 1.7610x; 1.7610x over previous
import functools
import jax
import jax.numpy as jnp
from jax import lax
from jax.experimental import pallas as pl
from jax.experimental.pallas import tpu as pltpu
from jax.experimental.pallas import tpu_sc as plsc

B, N, F = 4, 8192, 2048
NW = 32
WPB = NW // B
FW = F // WPB
R = 64
NT = N // R
NV = FW // 16
NS = 4

_mesh = plsc.VectorSubcoreMesh(core_axis_name="c", subcore_axis_name="s")


@functools.partial(
    pl.kernel,
    mesh=_mesh,
    out_type=jax.ShapeDtypeStruct((B, N, F), jnp.float32),
    scratch_types=[
        pltpu.VMEM((NS, R, FW), jnp.float32),
        pltpu.SemaphoreType.DMA,
        pltpu.SemaphoreType.DMA,
        pltpu.SemaphoreType.DMA,
        pltpu.SemaphoreType.DMA,
        pltpu.SemaphoreType.DMA,
        pltpu.SemaphoreType.DMA,
        pltpu.SemaphoreType.DMA,
        pltpu.SemaphoreType.DMA,
    ],
)
def _sc_cumsum(x_hbm, out_hbm, buf, l0, l1, l2, l3, s0, s1, s2, s3):
    wid = lax.axis_index("s") * 2 + lax.axis_index("c")
    b = wid // WPB
    f0 = (wid % WPB) * FW
    lsems = [l0, l1, l2, l3]
    ssems = [s0, s1, s2, s3]

    def load_copy(t, s):
        return pltpu.make_async_copy(
            x_hbm.at[b, pl.ds(t * R, R), pl.ds(f0, FW)],
            buf.at[s],
            lsems[s],
        )

    def store_copy(t, s):
        return pltpu.make_async_copy(
            buf.at[s],
            out_hbm.at[b, pl.ds(t * R, R), pl.ds(f0, FW)],
            ssems[s],
        )

    load_copy(0, 0).start()
    load_copy(1, 1).start()

    def phase(t, s, carry):
        load_copy(t, s).wait()

        def row(r, acc):
            new = []
            for j in range(NV):
                v = acc[j] + buf[s, r, pl.ds(16 * j, 16)]
                buf[s, r, pl.ds(16 * j, 16)] = v
                new.append(v)
            return tuple(new)

        carry = lax.fori_loop(0, R, row, carry, unroll=2)
        store_copy(t, s).start()

        s2_ = (s + 2) % NS

        @pl.when(t + 2 < NT)
        def _():
            @pl.when(t >= 2)
            def _():
                store_copy(t - 2, s2_).wait()

            load_copy(t + 2, s2_).start()

        return carry

    def four(i, carry):
        t = i * NS
        for k in range(NS):
            carry = phase(t + k, k, carry)
        return carry

    zeros = tuple(jnp.zeros((16,), jnp.float32) for _ in range(NV))
    lax.fori_loop(0, NT // NS, four, zeros)

    store_copy(NT - 2, (NT - 2) % NS).wait()
    store_copy(NT - 1, (NT - 1) % NS).wait()


def kernel(x):
    return _sc_cumsum(x)
